# baseline (speedup 1.0000x reference)
import functools

import jax
import jax.numpy as jnp
from jax import lax
from jax.experimental import pallas as pl
from jax.experimental.pallas import tpu as pltpu
from jax.experimental.pallas import tpu_sc as plsc

EMB = 64
NC = 2
NS = 16
NW = NC * NS
BLK = 128
NBUF = 4

_mesh = plsc.VectorSubcoreMesh(core_axis_name="c", subcore_axis_name="s")


def _make_gather(seq: int):
    et = EMB // 8

    @functools.partial(
        pl.kernel,
        mesh=_mesh,
        out_type=jax.ShapeDtypeStruct((seq, et, NW, 8, BLK), jnp.float32),
        scratch_types=[
            pltpu.VMEM((seq, BLK), jnp.int32),
            [pltpu.VMEM((BLK, EMB), jnp.float32) for _ in range(NBUF)],
            [pltpu.VMEM((et, 8, BLK), jnp.float32) for _ in range(NBUF)],
            [pltpu.SemaphoreType.DMA for _ in range(NBUF)],
            [pltpu.SemaphoreType.DMA for _ in range(NBUF)],
        ],
        compiler_params=pltpu.CompilerParams(
            use_tc_tiling_on_sc=False, needs_layout_passes=False
        ),
    )
    def gather_kernel(ids_hbm, table_hbm, out_hbm, idx_v, rbufs, tbufs, gsems, osems):
        wid = lax.axis_index("s") * NC + lax.axis_index("c")

        pltpu.sync_copy(ids_hbm.at[:, pl.ds(wid * BLK, BLK)], idx_v)

        rowidx = [lax.iota(jnp.int32, 16) + blk * 16 for blk in range(8)]

        def transpose(rb, tb):
            def col(c, _):
                colvec = jnp.full((16,), c, jnp.int32)
                e8 = c // 8
                el = c % 8
                for blk in range(8):
                    v = plsc.load_gather(rb, [rowidx[blk], colvec])
                    tb[e8, el, pl.ds(blk * 16, 16)] = v
                return _

            lax.fori_loop(0, EMB, col, None)

        def step(i, b, wait_out, refill):
            if wait_out:
                pltpu.make_async_copy(
                    tbufs[b], out_hbm.at[0, :, wid], osems[b]
                ).wait()
            pltpu.make_async_copy(
                table_hbm.at[idx_v.at[i]], rbufs[b], gsems[b]
            ).wait()
            transpose(rbufs[b], tbufs[b])
            pltpu.async_copy(tbufs[b], out_hbm.at[i, :, wid], osems[b])
            if refill:
                pltpu.async_copy(
                    table_hbm.at[idx_v.at[i + NBUF]], rbufs[b], gsems[b]
                )

        for b in range(NBUF):
            pltpu.async_copy(table_hbm.at[idx_v.at[b]], rbufs[b], gsems[b])

        for b in range(NBUF):
            step(b, b, wait_out=False, refill=True)

        def group(g, _):
            for b in range(NBUF):
                step(g * NBUF + b, b, wait_out=True, refill=True)
            return _

        lax.fori_loop(1, seq // NBUF - 1, group, None)

        for b in range(NBUF):
            step(seq - NBUF + b, b, wait_out=True, refill=False)

        for b in range(NBUF):
            pltpu.make_async_copy(
                tbufs[b], out_hbm.at[0, :, wid], osems[b]
            ).wait()

    return gather_kernel


def kernel(input_ids, token_table):
    batch, seq = input_ids.shape
    ids_t = jnp.transpose(input_ids.astype(jnp.int32))
    out5 = _make_gather(seq)(ids_t, token_table)
    return jnp.transpose(out5, (2, 4, 0, 1, 3)).reshape(batch, seq, EMB)
